# baseline (speedup 1.0000x reference)
import jax
import jax.numpy as jnp
from jax.experimental import pallas as pl

TEMP = 5.0


def _gin(x, src, dst, ew, layers):
    h = x
    for (W1, b1, W2, b2, eps) in layers:
        msg = h[src] * ew[:, None]
        agg = jax.ops.segment_sum(msg, dst, num_segments=x.shape[0])
        z = (1.0 + eps) * h + agg
        h = jax.nn.relu(jax.nn.relu(z @ W1 + b1) @ W2 + b2)
    return h


def kernel(x, edge_index, batch, params):
    src = edge_index[0]
    dst = edge_index[1]
    ones = jnp.ones((src.shape[0],), x.dtype)
    caus = [(l["W1"], l["b1"], l["W2"], l["b2"], l["eps"]) for l in params["causal"]]
    Z = _gin(x, src, dst, ones, caus)
    edge_feat = jnp.concatenate([Z[src], Z[dst]], axis=1)
    outs = []
    K = len(params["mask"])
    G = 128
    for k in range(K):
        m = params["mask"][k]
        eml = jax.nn.relu(edge_feat @ m["W1"] + m["b1"]) @ m["W2"] + m["b2"]
        emask = jax.nn.sigmoid(eml / TEMP).reshape(-1)
        layers = [(l["W1"], l["b1"], l["W2"], l["b2"], l["eps"]) for l in params["clf_enc"][k]]
        h = _gin(x, src, dst, emask, layers)
        hs = jax.ops.segment_sum(h, batch, num_segments=G)
        cnt = jax.ops.segment_sum(jnp.ones((x.shape[0],), x.dtype), batch, num_segments=G)
        hg = hs / jnp.clip(cnt, 1.0)[:, None]
        c = params["clf_head"][k]
        outs.append(jax.nn.relu(hg @ c["W1"] + c["b1"]) @ c["W2"] + c["b2"])
    return jnp.stack(outs, axis=0)


# speedup vs baseline: 2.6319x; 2.6319x over previous
import functools

import jax
import jax.numpy as jnp
from jax import lax
from jax.experimental import pallas as pl
from jax.experimental.pallas import tpu as pltpu
from jax.experimental.pallas import tpu_sc as plsc

N = 10000
NP = 10240
F = 128
H = 64
G = 128
C = 2
TEMP = 5.0
CH = 128
NC = 2
NS = 16
EP = 327680
ECH = EP // CH
BLK = 1024
NB = NP // BLK

_mesh = plsc.VectorSubcoreMesh(core_axis_name="c", subcore_axis_name="s")
_sc_params = pltpu.CompilerParams(use_tc_tiling_on_sc=False,
                                 needs_layout_passes=False)


def _zero_rows(ref, nrows, d):
    z = jnp.zeros((16,), jnp.float32)

    def body(i, carry):
        for f in range(d // 16):
            ref[i, pl.ds(f * 16, 16)] = z
        return carry

    lax.fori_loop(0, nrows, body, 0)


def _sc_seg_unweighted(d):
    EPW = EP // (NC * NS)
    NCHW = EPW // CH
    RPT = NP // NS

    @functools.partial(
        pl.kernel,
        out_type=jax.ShapeDtypeStruct((NC * NP, d), jnp.float32),
        mesh=_mesh,
        compiler_params=_sc_params,
        scratch_types=[
            pltpu.VMEM((8, CH), jnp.int32),
            pltpu.VMEM((8, CH), jnp.int32),
            pltpu.VMEM((CH, d), jnp.float32),
            pltpu.VMEM_SHARED((NP, d), jnp.float32),
            pltpu.SemaphoreType.DMA,
        ],
    )
    def k(h_hbm, src_hbm, dst_hbm, out_hbm, src_v, dst_v, rows_v, acc_sh, sem):
        c = lax.axis_index("c")
        s = lax.axis_index("s")
        wid = c * NS + s

        _zero_rows(rows_v, CH, d)

        def zacc(t, carry):
            pltpu.sync_copy(rows_v, acc_sh.at[pl.ds(s * RPT + t * CH, CH)])
            return carry

        lax.fori_loop(0, RPT // CH, zacc, 0)
        plsc.subcore_barrier()

        def superchunk(u, carry):
            pltpu.sync_copy(src_hbm.at[pl.ds(wid * NCHW + u * 8, 8)], src_v)
            pltpu.sync_copy(dst_hbm.at[pl.ds(wid * NCHW + u * 8, 8)], dst_v)

            def chunk(j, jcarry):
                pltpu.async_copy(h_hbm.at[src_v.at[j]], rows_v, sem).wait()
                pltpu.sync_copy(rows_v, acc_sh.at[dst_v.at[j]], add=True)
                return jcarry

            lax.fori_loop(0, 8, chunk, 0)
            return carry

        lax.fori_loop(0, NCHW // 8, superchunk, 0)
        plsc.subcore_barrier()

        def cout(t, carry):
            pltpu.sync_copy(acc_sh.at[pl.ds(s * RPT + t * CH, CH)], rows_v)
            pltpu.sync_copy(rows_v, out_hbm.at[pl.ds(c * NP + s * RPT + t * CH, CH)])
            return carry

        lax.fori_loop(0, RPT // CH, cout, 0)

    return k


def _sc_mask():
    EPW = EP // (NC * NS)
    NCHW = EPW // CH
    GR = CH // 16
    EMR = EP // 16

    @functools.partial(
        pl.kernel,
        out_type=jax.ShapeDtypeStruct((2 * EMR, 16), jnp.float32),
        mesh=_mesh,
        compiler_params=_sc_params,
        scratch_types=[
            pltpu.VMEM((NCHW, CH), jnp.int32),
            pltpu.VMEM((NCHW, CH), jnp.int32),
            pltpu.VMEM((CH, 2 * H), jnp.float32),
            pltpu.VMEM((CH, 2 * H), jnp.float32),
            pltpu.VMEM((2, H), jnp.float32),
            pltpu.VMEM((2, H), jnp.float32),
            pltpu.VMEM((2, 16), jnp.float32),
            pltpu.VMEM((2 * GR, 16), jnp.float32),
            pltpu.SemaphoreType.DMA,
            pltpu.SemaphoreType.DMA,
        ],
    )
    def k(p_hbm, q_hbm, src_hbm, dst_hbm, b1_hbm, w2_hbm, b2_hbm, em_hbm,
          src_v, dst_v, prow, qrow, b1v, w2v, b2v, embuf, sem_p, sem_q):
        c = lax.axis_index("c")
        s = lax.axis_index("s")
        wid = c * NS + s

        pltpu.sync_copy(b1_hbm, b1v)
        pltpu.sync_copy(w2_hbm, w2v)
        pltpu.sync_copy(b2_hbm, b2v)
        pltpu.sync_copy(src_hbm.at[pl.ds(wid * NCHW, NCHW)], src_v)
        pltpu.sync_copy(dst_hbm.at[pl.ds(wid * NCHW, NCHW)], dst_v)

        b1r = [[b1v[kk, pl.ds(f * 16, 16)] for f in range(4)] for kk in range(2)]
        w2r = [[w2v[kk, pl.ds(f * 16, 16)] for f in range(4)] for kk in range(2)]
        b2s = [b2v[kk][0] for kk in range(2)]
        lane = lax.iota(jnp.int32, 16)

        def chunk(j, carry):
            cp_p = pltpu.async_copy(p_hbm.at[src_v.at[j]], prow, sem_p)
            cp_q = pltpu.async_copy(q_hbm.at[dst_v.at[j]], qrow, sem_q)
            cp_p.wait()
            cp_q.wait()

            def group(g, gcarry):
                lv = [jnp.zeros((16,), jnp.float32) for _ in range(2)]
                for l in range(16):
                    e = g * 16 + l
                    for kk in range(2):
                        t = None
                        for f4 in range(4):
                            f = kk * 4 + f4
                            pv = prow[e, pl.ds(f * 16, 16)]
                            qv = qrow[e, pl.ds(f * 16, 16)]
                            sv = jnp.maximum(pv + qv + b1r[kk][f4], 0.0)
                            tv = sv * w2r[kk][f4]
                            t = tv if t is None else t + tv
                        lgt = jnp.sum(t) + b2s[kk]
                        lv[kk] = jnp.where(lane == l, lgt, lv[kk])
                for kk in range(2):
                    embuf[kk * GR + g, :] = 1.0 / (1.0 + jnp.exp(-lv[kk]))
                return gcarry

            lax.fori_loop(0, GR, group, 0)
            for kk in range(2):
                pltpu.sync_copy(
                    embuf.at[pl.ds(kk * GR, GR)],
                    em_hbm.at[pl.ds(kk * EMR + wid * (EPW // 16) + j * GR, GR)])
            return carry

        lax.fori_loop(0, NCHW, chunk, 0)

    return k


def _sc_expert0():
    EPW = EP // NS
    NCHW = EPW // CH
    RPT = NP // NS

    @functools.partial(
        pl.kernel,
        out_type=jax.ShapeDtypeStruct((NC * 2 * NP, H), jnp.float32),
        mesh=_mesh,
        compiler_params=_sc_params,
        scratch_types=[
            pltpu.VMEM((8, CH), jnp.int32),
            pltpu.VMEM((8, CH), jnp.int32),
            pltpu.VMEM((8 * (CH // 16), 16), jnp.float32),
            pltpu.VMEM((8 * (CH // 16), 16), jnp.float32),
            pltpu.VMEM((CH, H), jnp.float32),
            pltpu.VMEM((CH, H), jnp.float32),
            pltpu.VMEM((CH, H), jnp.float32),
            pltpu.VMEM_SHARED((2, NP, H), jnp.float32),
            pltpu.SemaphoreType.DMA,
        ],
    )
    def k(xh_hbm, src_hbm, dst_hbm, em_hbm, out_hbm,
          src_v, dst_v, em0_v, em1_v, rows_v, st0, st1, acc_sh, sem):
        c = lax.axis_index("c")
        s = lax.axis_index("s")
        GR = CH // 16
        EMR = EP // 16

        _zero_rows(st0, CH, H)
        def zacc(t, carry):
            for kk in range(2):
                pltpu.sync_copy(st0, acc_sh.at[kk, pl.ds(s * RPT + t * CH, CH)])
            return carry
        lax.fori_loop(0, RPT // CH, zacc, 0)
        plsc.subcore_barrier()

        off = c * NP

        def superchunk(u, carry):
            pltpu.sync_copy(src_hbm.at[pl.ds(s * NCHW + u * 8, 8)], src_v)
            pltpu.sync_copy(dst_hbm.at[pl.ds(s * NCHW + u * 8, 8)], dst_v)
            pltpu.sync_copy(
                em_hbm.at[pl.ds(s * (EPW // 16) + u * 8 * GR, 8 * GR)], em0_v)
            pltpu.sync_copy(
                em_hbm.at[pl.ds(EMR + s * (EPW // 16) + u * 8 * GR, 8 * GR)], em1_v)

            def adj(i, icarry):
                for f in range(CH // 16):
                    src_v[i, pl.ds(f * 16, 16)] = src_v[i, pl.ds(f * 16, 16)] + off
                return icarry
            lax.fori_loop(0, 8, adj, 0)

            def chunk(j, jcarry):
                pltpu.async_copy(xh_hbm.at[src_v.at[j]], rows_v, sem).wait()

                def group(g, gcarry):
                    mv0 = em0_v[j * GR + g]
                    mv1 = em1_v[j * GR + g]
                    for l in range(16):
                        e = g * 16 + l
                        m0 = mv0[l]
                        m1 = mv1[l]
                        for f in range(H // 16):
                            rv = rows_v[e, pl.ds(f * 16, 16)]
                            st0[e, pl.ds(f * 16, 16)] = rv * m0
                            st1[e, pl.ds(f * 16, 16)] = rv * m1
                    return gcarry

                lax.fori_loop(0, GR, group, 0)
                pltpu.sync_copy(st0, acc_sh.at[0].at[dst_v.at[j]], add=True)
                pltpu.sync_copy(st1, acc_sh.at[1].at[dst_v.at[j]], add=True)
                return jcarry

            lax.fori_loop(0, 8, chunk, 0)
            return carry

        lax.fori_loop(0, NCHW // 8, superchunk, 0)
        plsc.subcore_barrier()

        def cout(t, carry):
            for kk in range(2):
                pltpu.sync_copy(acc_sh.at[kk, pl.ds(s * RPT + t * CH, CH)], st0)
                pltpu.sync_copy(
                    st0,
                    out_hbm.at[pl.ds((c * 2 + kk) * NP + s * RPT + t * CH, CH)])
            return carry

        lax.fori_loop(0, RPT // CH, cout, 0)

    return k


def _sc_expert1():
    EPW = EP // NS
    NCHW = EPW // CH
    RPT = NP // NS

    @functools.partial(
        pl.kernel,
        out_type=jax.ShapeDtypeStruct((NC * NP, H), jnp.float32),
        mesh=_mesh,
        compiler_params=_sc_params,
        scratch_types=[
            pltpu.VMEM((NCHW, CH), jnp.int32),
            pltpu.VMEM((NCHW, CH), jnp.int32),
            pltpu.VMEM((NCHW * (CH // 16), 16), jnp.float32),
            pltpu.VMEM((CH, H), jnp.float32),
            pltpu.VMEM_SHARED((NP, H), jnp.float32),
            pltpu.SemaphoreType.DMA,
        ],
    )
    def k(h_hbm, src_hbm, dst_hbm, em_hbm, out_hbm,
          src_v, dst_v, em_v, rows_v, acc_sh, sem):
        c = lax.axis_index("c")
        s = lax.axis_index("s")
        GR = CH // 16
        EMR = EP // 16

        _zero_rows(rows_v, CH, H)
        def zacc(t, carry):
            pltpu.sync_copy(rows_v, acc_sh.at[pl.ds(s * RPT + t * CH, CH)])
            return carry
        lax.fori_loop(0, RPT // CH, zacc, 0)
        plsc.subcore_barrier()

        pltpu.sync_copy(src_hbm.at[pl.ds(s * NCHW, NCHW)], src_v)
        pltpu.sync_copy(dst_hbm.at[pl.ds(s * NCHW, NCHW)], dst_v)
        pltpu.sync_copy(em_hbm.at[pl.ds(c * EMR + s * (EPW // 16), NCHW * GR)], em_v)

        off = c * NP
        def adj(i, carry):
            for f in range(CH // 16):
                src_v[i, pl.ds(f * 16, 16)] = src_v[i, pl.ds(f * 16, 16)] + off
            return carry
        lax.fori_loop(0, NCHW, adj, 0)

        def chunk(j, carry):
            pltpu.async_copy(h_hbm.at[src_v.at[j]], rows_v, sem).wait()

            def group(g, gcarry):
                mv = em_v[j * GR + g]
                for l in range(16):
                    e = g * 16 + l
                    m = mv[l]
                    for f in range(H // 16):
                        rows_v[e, pl.ds(f * 16, 16)] = rows_v[e, pl.ds(f * 16, 16)] * m
                return gcarry

            lax.fori_loop(0, GR, group, 0)
            pltpu.sync_copy(rows_v, acc_sh.at[dst_v.at[j]], add=True)
            return carry

        lax.fori_loop(0, NCHW, chunk, 0)
        plsc.subcore_barrier()

        def cout(t, carry):
            pltpu.sync_copy(acc_sh.at[pl.ds(s * RPT + t * CH, CH)], rows_v)
            pltpu.sync_copy(rows_v, out_hbm.at[pl.ds(c * NP + s * RPT + t * CH, CH)])
            return carry

        lax.fori_loop(0, RPT // CH, cout, 0)

    return k


def _tc_mlp(h, parts, W1, b1, W2, b2, epsv):
    din = h.shape[1]

    def body(h_ref, p_ref, W1_ref, b1_ref, W2_ref, b2_ref, eps_ref, o_ref):
        eps = eps_ref[0, 0]
        z = (1.0 + eps) * h_ref[...] + p_ref[0] + p_ref[1]
        y = jnp.maximum(jnp.dot(z, W1_ref[...], preferred_element_type=jnp.float32,
                        precision=lax.Precision.HIGHEST)
                        + b1_ref[...], 0.0)
        o_ref[...] = jnp.maximum(
            jnp.dot(y, W2_ref[...], preferred_element_type=jnp.float32,
                        precision=lax.Precision.HIGHEST)
            + b2_ref[...], 0.0)

    return pl.pallas_call(
        body,
        grid=(NB,),
        in_specs=[
            pl.BlockSpec((BLK, din), lambda i: (i, 0)),
            pl.BlockSpec((2, BLK, din), lambda i: (0, i, 0)),
            pl.BlockSpec((din, H), lambda i: (0, 0)),
            pl.BlockSpec((1, H), lambda i: (0, 0)),
            pl.BlockSpec((H, H), lambda i: (0, 0)),
            pl.BlockSpec((1, H), lambda i: (0, 0)),
            pl.BlockSpec((1, 128), lambda i: (0, 0)),
        ],
        out_specs=pl.BlockSpec((BLK, H), lambda i: (i, 0)),
        out_shape=jax.ShapeDtypeStruct((NP, H), jnp.float32),
    )(h, parts, W1, b1, W2, b2, epsv)


def _tc_mlp_pq(h, parts, W1, b1, W2, b2, epsv, Wp, Wq):
    din = h.shape[1]

    def body(h_ref, p_ref, W1_ref, b1_ref, W2_ref, b2_ref, eps_ref,
             Wp_ref, Wq_ref, z_ref, pp_ref, qq_ref):
        eps = eps_ref[0, 0]
        z = (1.0 + eps) * h_ref[...] + p_ref[0] + p_ref[1]
        y = jnp.maximum(jnp.dot(z, W1_ref[...], preferred_element_type=jnp.float32,
                        precision=lax.Precision.HIGHEST)
                        + b1_ref[...], 0.0)
        Z = jnp.maximum(jnp.dot(y, W2_ref[...], preferred_element_type=jnp.float32,
                        precision=lax.Precision.HIGHEST)
                        + b2_ref[...], 0.0)
        z_ref[...] = Z
        pp_ref[...] = jnp.dot(Z, Wp_ref[...], preferred_element_type=jnp.float32,
                        precision=lax.Precision.HIGHEST)
        qq_ref[...] = jnp.dot(Z, Wq_ref[...], preferred_element_type=jnp.float32,
                        precision=lax.Precision.HIGHEST)

    return pl.pallas_call(
        body,
        grid=(NB,),
        in_specs=[
            pl.BlockSpec((BLK, din), lambda i: (i, 0)),
            pl.BlockSpec((2, BLK, din), lambda i: (0, i, 0)),
            pl.BlockSpec((din, H), lambda i: (0, 0)),
            pl.BlockSpec((1, H), lambda i: (0, 0)),
            pl.BlockSpec((H, H), lambda i: (0, 0)),
            pl.BlockSpec((1, H), lambda i: (0, 0)),
            pl.BlockSpec((1, 128), lambda i: (0, 0)),
            pl.BlockSpec((H, 128), lambda i: (0, 0)),
            pl.BlockSpec((H, 128), lambda i: (0, 0)),
        ],
        out_specs=[
            pl.BlockSpec((BLK, H), lambda i: (i, 0)),
            pl.BlockSpec((BLK, 128), lambda i: (i, 0)),
            pl.BlockSpec((BLK, 128), lambda i: (i, 0)),
        ],
        out_shape=[
            jax.ShapeDtypeStruct((NP, H), jnp.float32),
            jax.ShapeDtypeStruct((NP, 128), jnp.float32),
            jax.ShapeDtypeStruct((NP, 128), jnp.float32),
        ],
    )(h, parts, W1, b1, W2, b2, epsv, Wp, Wq)


def _tc_expert_mlp(x, agg4, W1s, b1s, W2s, b2s, epsv):
    def body(x_ref, a_ref, W1_ref, b1_ref, W2_ref, b2_ref, eps_ref, o_ref):
        eps = eps_ref[0, 0, 0]
        agg = jnp.concatenate([a_ref[0, 0], a_ref[1, 0]], axis=-1)
        z = (1.0 + eps) * x_ref[...] + agg
        y = jnp.maximum(jnp.dot(z, W1_ref[0], preferred_element_type=jnp.float32,
                        precision=lax.Precision.HIGHEST)
                        + b1_ref[0], 0.0)
        o_ref[0] = jnp.maximum(
            jnp.dot(y, W2_ref[0], preferred_element_type=jnp.float32,
                        precision=lax.Precision.HIGHEST)
            + b2_ref[0], 0.0)

    return pl.pallas_call(
        body,
        grid=(2, NB),
        in_specs=[
            pl.BlockSpec((BLK, F), lambda k, i: (i, 0)),
            pl.BlockSpec((2, 1, BLK, H), lambda k, i: (0, k, i, 0)),
            pl.BlockSpec((1, F, H), lambda k, i: (k, 0, 0)),
            pl.BlockSpec((1, 1, H), lambda k, i: (k, 0, 0)),
            pl.BlockSpec((1, H, H), lambda k, i: (k, 0, 0)),
            pl.BlockSpec((1, 1, H), lambda k, i: (k, 0, 0)),
            pl.BlockSpec((1, 1, 128), lambda k, i: (k, 0, 0)),
        ],
        out_specs=pl.BlockSpec((1, BLK, H), lambda k, i: (k, i, 0)),
        out_shape=jax.ShapeDtypeStruct((2, NP, H), jnp.float32),
    )(x, agg4, W1s, b1s, W2s, b2s, epsv)


def _tc_final(h1e, agg5, batch3d, W1s, b1s, W2s, b2s, epsv, Wc1, bc1, Wc2, bc2):
    def body(h_ref, a_ref, b_ref, W1_ref, b1_ref, W2_ref, b2_ref, eps_ref,
             Wc1_ref, bc1_ref, Wc2_ref, bc2_ref, o_ref, hs_ref, cnt_ref):
        i = pl.program_id(1)
        eps = eps_ref[0, 0, 0]
        z = (1.0 + eps) * h_ref[0] + a_ref[0]
        y = jnp.maximum(jnp.dot(z, W1_ref[0], preferred_element_type=jnp.float32,
                        precision=lax.Precision.HIGHEST)
                        + b1_ref[0], 0.0)
        h2 = jnp.maximum(jnp.dot(y, W2_ref[0], preferred_element_type=jnp.float32,
                        precision=lax.Precision.HIGHEST)
                         + b2_ref[0], 0.0)
        b = b_ref[0, 0]
        gidx = lax.broadcasted_iota(jnp.int32, (G, BLK), 0)
        m = (b[None, :] == gidx).astype(jnp.float32)
        pooled = jnp.dot(m, h2, preferred_element_type=jnp.float32,
                        precision=lax.Precision.HIGHEST)
        csum = jnp.broadcast_to(jnp.sum(m, axis=1, keepdims=True), (G, 128))

        @pl.when(i == 0)
        def _():
            hs_ref[...] = pooled
            cnt_ref[...] = csum

        @pl.when(i > 0)
        def _():
            hs_ref[...] = hs_ref[...] + pooled
            cnt_ref[...] = cnt_ref[...] + csum

        @pl.when(i == NB - 1)
        def _():
            hg = hs_ref[...] / jnp.maximum(cnt_ref[...][:, :H], 1.0)
            o1 = jnp.maximum(
                jnp.dot(hg, Wc1_ref[0], preferred_element_type=jnp.float32,
                        precision=lax.Precision.HIGHEST)
                + bc1_ref[0], 0.0)
            o_ref[0] = jnp.dot(o1, Wc2_ref[0], preferred_element_type=jnp.float32,
                        precision=lax.Precision.HIGHEST) + bc2_ref[0]

    return pl.pallas_call(
        body,
        grid=(2, NB),
        in_specs=[
            pl.BlockSpec((1, BLK, H), lambda k, i: (k, i, 0)),
            pl.BlockSpec((1, BLK, H), lambda k, i: (k, i, 0)),
            pl.BlockSpec((1, 1, BLK), lambda k, i: (i, 0, 0)),
            pl.BlockSpec((1, H, H), lambda k, i: (k, 0, 0)),
            pl.BlockSpec((1, 1, H), lambda k, i: (k, 0, 0)),
            pl.BlockSpec((1, H, H), lambda k, i: (k, 0, 0)),
            pl.BlockSpec((1, 1, H), lambda k, i: (k, 0, 0)),
            pl.BlockSpec((1, 1, 128), lambda k, i: (k, 0, 0)),
            pl.BlockSpec((1, H, H), lambda k, i: (k, 0, 0)),
            pl.BlockSpec((1, 1, H), lambda k, i: (k, 0, 0)),
            pl.BlockSpec((1, H, 128), lambda k, i: (k, 0, 0)),
            pl.BlockSpec((1, 1, 128), lambda k, i: (k, 0, 0)),
        ],
        out_specs=pl.BlockSpec((1, G, 128), lambda k, i: (k, 0, 0)),
        out_shape=jax.ShapeDtypeStruct((2, G, 128), jnp.float32),
        scratch_shapes=[
            pltpu.VMEM((G, H), jnp.float32),
            pltpu.VMEM((G, 128), jnp.float32),
        ],
    )(h1e, agg5, batch3d, W1s, b1s, W2s, b2s, epsv, Wc1, bc1, Wc2, bc2)


def kernel(x, edge_index, batch, params):
    src = edge_index[0]
    dst = edge_index[1]
    E = src.shape[0]

    xp = jnp.pad(x, ((0, NP - N), (0, 0)))
    srcp = jnp.pad(src, (0, EP - E), constant_values=N).reshape(ECH, CH)
    dstp = jnp.pad(dst, (0, EP - E), constant_values=N).reshape(ECH, CH)
    batch3d = jnp.pad(batch, (0, NP - N), constant_values=G).reshape(NB, 1, BLK)

    cl0, cl1 = params["causal"]

    def epsv(e):
        return jnp.broadcast_to(e, (1, 128)).astype(jnp.float32)

    p1 = _sc_seg_unweighted(F)(xp, srcp, dstp).reshape(2, NP, F)
    h1 = _tc_mlp(xp, p1, cl0["W1"], cl0["b1"].reshape(1, H), cl0["W2"],
                 cl0["b2"].reshape(1, H), epsv(cl0["eps"]))
    p2 = _sc_seg_unweighted(H)(h1, srcp, dstp).reshape(2, NP, H)

    m0, m1 = params["mask"][0], params["mask"][1]
    Wp = jnp.concatenate([m0["W1"][:H], m1["W1"][:H]], axis=1)
    Wq = jnp.concatenate([m0["W1"][H:], m1["W1"][H:]], axis=1)
    Z, P, Q = _tc_mlp_pq(h1, p2, cl1["W1"], cl1["b1"].reshape(1, H), cl1["W2"],
                         cl1["b2"].reshape(1, H), epsv(cl1["eps"]), Wp, Wq)

    b1all = jnp.stack([m0["b1"], m1["b1"]])
    w2all = jnp.stack([m0["W2"][:, 0], m1["W2"][:, 0]]) / TEMP
    b2all = jnp.broadcast_to(
        (jnp.stack([m0["b2"][0], m1["b2"][0]]) / TEMP)[:, None], (2, 16))

    em = _sc_mask()(P, Q, srcp, dstp, b1all, w2all, b2all)

    xh = jnp.concatenate([xp[:, :H], xp[:, H:]], axis=0)
    agg4 = _sc_expert0()(xh, srcp, dstp, em).reshape(2, 2, NP, H)

    enc = params["clf_enc"]
    W1s0 = jnp.stack([enc[k][0]["W1"] for k in range(2)])
    b1s0 = jnp.stack([enc[k][0]["b1"].reshape(1, H) for k in range(2)])
    W2s0 = jnp.stack([enc[k][0]["W2"] for k in range(2)])
    b2s0 = jnp.stack([enc[k][0]["b2"].reshape(1, H) for k in range(2)])
    eps0 = jnp.stack([jnp.broadcast_to(enc[k][0]["eps"], (1, 128)) for k in range(2)])
    h1e = _tc_expert_mlp(xp, agg4, W1s0, b1s0, W2s0, b2s0,
                         eps0.astype(jnp.float32))

    h1e_flat = h1e.reshape(2 * NP, H)
    agg5 = _sc_expert1()(h1e_flat, srcp, dstp, em).reshape(2, NP, H)

    W1s1 = jnp.stack([enc[k][1]["W1"] for k in range(2)])
    b1s1 = jnp.stack([enc[k][1]["b1"].reshape(1, H) for k in range(2)])
    W2s1 = jnp.stack([enc[k][1]["W2"] for k in range(2)])
    b2s1 = jnp.stack([enc[k][1]["b2"].reshape(1, H) for k in range(2)])
    eps1 = jnp.stack([jnp.broadcast_to(enc[k][1]["eps"], (1, 128)) for k in range(2)])

    hd = params["clf_head"]
    Wc1 = jnp.stack([hd[k]["W1"] for k in range(2)])
    bc1 = jnp.stack([hd[k]["b1"].reshape(1, H) for k in range(2)])
    Wc2 = jnp.stack([jnp.pad(hd[k]["W2"], ((0, 0), (0, 128 - C))) for k in range(2)])
    bc2 = jnp.stack([jnp.pad(jnp.broadcast_to(hd[k]["b2"], (C,)),
                             (0, 128 - C)).reshape(1, 128) for k in range(2)])

    out = _tc_final(h1e, agg5, batch3d, W1s1, b1s1, W2s1, b2s1,
                    eps1.astype(jnp.float32), Wc1, bc1, Wc2, bc2)
    return out[:, :, :C]


# speedup vs baseline: 2.6335x; 1.0006x over previous
import functools

import jax
import jax.numpy as jnp
from jax import lax
from jax.experimental import pallas as pl
from jax.experimental.pallas import tpu as pltpu
from jax.experimental.pallas import tpu_sc as plsc

N = 10000
NP = 10240
F = 128
H = 64
G = 128
C = 2
TEMP = 5.0
CH = 128
NC = 2
NS = 16
EP = 327680
ECH = EP // CH
BLK = 1024
NB = NP // BLK

_mesh = plsc.VectorSubcoreMesh(core_axis_name="c", subcore_axis_name="s")
_sc_params = pltpu.CompilerParams(use_tc_tiling_on_sc=False,
                                 needs_layout_passes=False)


def _zero_rows(ref, nrows, d):
    z = jnp.zeros((16,), jnp.float32)

    def body(i, carry):
        for f in range(d // 16):
            ref[i, pl.ds(f * 16, 16)] = z
        return carry

    lax.fori_loop(0, nrows, body, 0)


def _sc_seg_unweighted(d):
    EPW = EP // (NC * NS)
    NCHW = EPW // CH
    RPT = NP // NS

    @functools.partial(
        pl.kernel,
        out_type=jax.ShapeDtypeStruct((NC * NP, d), jnp.float32),
        mesh=_mesh,
        compiler_params=_sc_params,
        scratch_types=[
            pltpu.VMEM((16, CH), jnp.int32),
            pltpu.VMEM((16, CH), jnp.int32),
            pltpu.VMEM((CH, d), jnp.float32),
            pltpu.VMEM((CH, d), jnp.float32),
            pltpu.VMEM_SHARED((NP, d), jnp.float32),
            pltpu.SemaphoreType.DMA,
            pltpu.SemaphoreType.DMA,
        ],
    )
    def k(h_hbm, src_hbm, dst_hbm, out_hbm, src_v, dst_v, rows_a, rows_b,
          acc_sh, sem_a, sem_b):
        c = lax.axis_index("c")
        s = lax.axis_index("s")
        wid = c * NS + s

        _zero_rows(rows_a, CH, d)

        def zacc(t, carry):
            pltpu.sync_copy(rows_a, acc_sh.at[pl.ds(s * RPT + t * CH, CH)])
            return carry

        lax.fori_loop(0, RPT // CH, zacc, 0)
        plsc.subcore_barrier()

        def superchunk(u, carry):
            pltpu.sync_copy(src_hbm.at[pl.ds(wid * NCHW + u * 16, 16)], src_v)
            pltpu.sync_copy(dst_hbm.at[pl.ds(wid * NCHW + u * 16, 16)], dst_v)

            def pair(p, pcarry):
                cp_a = pltpu.async_copy(h_hbm.at[src_v.at[2 * p]], rows_a, sem_a)
                cp_b = pltpu.async_copy(h_hbm.at[src_v.at[2 * p + 1]], rows_b, sem_b)
                cp_a.wait()
                pltpu.sync_copy(rows_a, acc_sh.at[dst_v.at[2 * p]], add=True)
                cp_b.wait()
                pltpu.sync_copy(rows_b, acc_sh.at[dst_v.at[2 * p + 1]], add=True)
                return pcarry

            lax.fori_loop(0, 8, pair, 0)
            return carry

        lax.fori_loop(0, NCHW // 16, superchunk, 0)
        plsc.subcore_barrier()

        def cout(t, carry):
            pltpu.sync_copy(acc_sh.at[pl.ds(s * RPT + t * CH, CH)], rows_a)
            pltpu.sync_copy(rows_a, out_hbm.at[pl.ds(c * NP + s * RPT + t * CH, CH)])
            return carry

        lax.fori_loop(0, RPT // CH, cout, 0)

    return k


def _sc_mask():
    EPW = EP // (NC * NS)
    NCHW = EPW // CH
    GR = CH // 16
    EMR = EP // 16

    @functools.partial(
        pl.kernel,
        out_type=jax.ShapeDtypeStruct((2 * EMR, 16), jnp.float32),
        mesh=_mesh,
        compiler_params=_sc_params,
        scratch_types=[
            pltpu.VMEM((NCHW, CH), jnp.int32),
            pltpu.VMEM((NCHW, CH), jnp.int32),
            pltpu.VMEM((CH, 2 * H), jnp.float32),
            pltpu.VMEM((CH, 2 * H), jnp.float32),
            pltpu.VMEM((CH, 2 * H), jnp.float32),
            pltpu.VMEM((CH, 2 * H), jnp.float32),
            pltpu.VMEM((2, H), jnp.float32),
            pltpu.VMEM((2, 16), jnp.float32),
            pltpu.VMEM((2 * 16, 16), jnp.float32),
            pltpu.VMEM((2 * GR, 16), jnp.float32),
            pltpu.SemaphoreType.DMA,
            pltpu.SemaphoreType.DMA,
            pltpu.SemaphoreType.DMA,
            pltpu.SemaphoreType.DMA,
        ],
    )
    def k(p_hbm, q_hbm, src_hbm, dst_hbm, w2_hbm, b2_hbm, em_hbm,
          src_v, dst_v, prow_a, qrow_a, prow_b, qrow_b, w2v, b2v, tbuf, embuf,
          sem_pa, sem_qa, sem_pb, sem_qb):
        c = lax.axis_index("c")
        s = lax.axis_index("s")
        wid = c * NS + s

        pltpu.sync_copy(w2_hbm, w2v)
        pltpu.sync_copy(b2_hbm, b2v)
        pltpu.sync_copy(src_hbm.at[pl.ds(wid * NCHW, NCHW)], src_v)
        pltpu.sync_copy(dst_hbm.at[pl.ds(wid * NCHW, NCHW)], dst_v)

        w2r = [[w2v[kk, pl.ds(f * 16, 16)] for f in range(4)] for kk in range(2)]
        b2s = [b2v[kk][0] for kk in range(2)]
        lane = lax.iota(jnp.int32, 16)
        cols = [jnp.full((16,), cc, jnp.int32) for cc in range(16)]

        def compute_chunk(j, prow, qrow):
            def group(g, gcarry):
                for l in range(16):
                    e = g * 16 + l
                    for kk in range(2):
                        t = None
                        for f4 in range(4):
                            f = kk * 4 + f4
                            pv = prow[e, pl.ds(f * 16, 16)]
                            qv = qrow[e, pl.ds(f * 16, 16)]
                            sv = jnp.maximum(pv + qv, 0.0)
                            tv = sv * w2r[kk][f4]
                            t = tv if t is None else t + tv
                        tbuf[kk * 16 + l, :] = t
                for kk in range(2):
                    lv = None
                    rowsel = lane + kk * 16
                    for cc in range(16):
                        colv = plsc.load_gather(tbuf, [rowsel, cols[cc]])
                        lv = colv if lv is None else lv + colv
                    lv = lv + b2s[kk]
                    embuf[kk * GR + g, :] = 1.0 / (1.0 + jnp.exp(-lv))
                return gcarry

            lax.fori_loop(0, GR, group, 0)
            for kk in range(2):
                pltpu.sync_copy(
                    embuf.at[pl.ds(kk * GR, GR)],
                    em_hbm.at[pl.ds(kk * EMR + wid * (EPW // 16) + j * GR, GR)])

        def pair(p, carry):
            cp_pa = pltpu.async_copy(p_hbm.at[src_v.at[2 * p]], prow_a, sem_pa)
            cp_qa = pltpu.async_copy(q_hbm.at[dst_v.at[2 * p]], qrow_a, sem_qa)
            cp_pb = pltpu.async_copy(p_hbm.at[src_v.at[2 * p + 1]], prow_b, sem_pb)
            cp_qb = pltpu.async_copy(q_hbm.at[dst_v.at[2 * p + 1]], qrow_b, sem_qb)
            cp_pa.wait()
            cp_qa.wait()
            compute_chunk(2 * p, prow_a, qrow_a)
            cp_pb.wait()
            cp_qb.wait()
            compute_chunk(2 * p + 1, prow_b, qrow_b)
            return carry

        lax.fori_loop(0, NCHW // 2, pair, 0)

    return k


def _sc_expert0():
    EPW = EP // NS
    NCHW = EPW // CH
    RPT = NP // NS

    @functools.partial(
        pl.kernel,
        out_type=jax.ShapeDtypeStruct((NC * 2 * NP, H), jnp.float32),
        mesh=_mesh,
        compiler_params=_sc_params,
        scratch_types=[
            pltpu.VMEM((16, CH), jnp.int32),
            pltpu.VMEM((16, CH), jnp.int32),
            pltpu.VMEM((16 * (CH // 16), 16), jnp.float32),
            pltpu.VMEM((16 * (CH // 16), 16), jnp.float32),
            pltpu.VMEM((CH, H), jnp.float32),
            pltpu.VMEM((CH, H), jnp.float32),
            pltpu.VMEM((CH, H), jnp.float32),
            pltpu.VMEM((CH, H), jnp.float32),
            pltpu.VMEM_SHARED((2, NP, H), jnp.float32),
            pltpu.SemaphoreType.DMA,
            pltpu.SemaphoreType.DMA,
        ],
    )
    def k(xh_hbm, src_hbm, dst_hbm, em_hbm, out_hbm,
          src_v, dst_v, em0_v, em1_v, rows_a, rows_b, st0, st1, acc_sh,
          sem_a, sem_b):
        c = lax.axis_index("c")
        s = lax.axis_index("s")
        GR = CH // 16
        EMR = EP // 16

        _zero_rows(st0, CH, H)
        def zacc(t, carry):
            for kk in range(2):
                pltpu.sync_copy(st0, acc_sh.at[kk, pl.ds(s * RPT + t * CH, CH)])
            return carry
        lax.fori_loop(0, RPT // CH, zacc, 0)
        plsc.subcore_barrier()

        off = c * NP

        def scale_scatter(j, rows_v):
            def group(g, gcarry):
                mv0 = em0_v[j * GR + g]
                mv1 = em1_v[j * GR + g]
                for l in range(16):
                    e = g * 16 + l
                    m0 = mv0[l]
                    m1 = mv1[l]
                    for f in range(H // 16):
                        rv = rows_v[e, pl.ds(f * 16, 16)]
                        st0[e, pl.ds(f * 16, 16)] = rv * m0
                        st1[e, pl.ds(f * 16, 16)] = rv * m1
                return gcarry

            lax.fori_loop(0, GR, group, 0)
            pltpu.sync_copy(st0, acc_sh.at[0].at[dst_v.at[j]], add=True)
            pltpu.sync_copy(st1, acc_sh.at[1].at[dst_v.at[j]], add=True)

        def superchunk(u, carry):
            pltpu.sync_copy(src_hbm.at[pl.ds(s * NCHW + u * 16, 16)], src_v)
            pltpu.sync_copy(dst_hbm.at[pl.ds(s * NCHW + u * 16, 16)], dst_v)
            pltpu.sync_copy(
                em_hbm.at[pl.ds(s * (EPW // 16) + u * 16 * GR, 16 * GR)], em0_v)
            pltpu.sync_copy(
                em_hbm.at[pl.ds(EMR + s * (EPW // 16) + u * 16 * GR, 16 * GR)], em1_v)

            def adj(i, icarry):
                for f in range(CH // 16):
                    src_v[i, pl.ds(f * 16, 16)] = src_v[i, pl.ds(f * 16, 16)] + off
                return icarry
            lax.fori_loop(0, 16, adj, 0)

            def pair(p, pcarry):
                cp_a = pltpu.async_copy(xh_hbm.at[src_v.at[2 * p]], rows_a, sem_a)
                cp_b = pltpu.async_copy(xh_hbm.at[src_v.at[2 * p + 1]], rows_b, sem_b)
                cp_a.wait()
                scale_scatter(2 * p, rows_a)
                cp_b.wait()
                scale_scatter(2 * p + 1, rows_b)
                return pcarry

            lax.fori_loop(0, 8, pair, 0)
            return carry

        lax.fori_loop(0, NCHW // 16, superchunk, 0)
        plsc.subcore_barrier()

        def cout(t, carry):
            for kk in range(2):
                pltpu.sync_copy(acc_sh.at[kk, pl.ds(s * RPT + t * CH, CH)], st0)
                pltpu.sync_copy(
                    st0,
                    out_hbm.at[pl.ds((c * 2 + kk) * NP + s * RPT + t * CH, CH)])
            return carry

        lax.fori_loop(0, RPT // CH, cout, 0)

    return k


def _sc_expert1():
    EPW = EP // NS
    NCHW = EPW // CH
    RPT = NP // NS

    @functools.partial(
        pl.kernel,
        out_type=jax.ShapeDtypeStruct((NC * NP, H), jnp.float32),
        mesh=_mesh,
        compiler_params=_sc_params,
        scratch_types=[
            pltpu.VMEM((NCHW, CH), jnp.int32),
            pltpu.VMEM((NCHW, CH), jnp.int32),
            pltpu.VMEM((NCHW * (CH // 16), 16), jnp.float32),
            pltpu.VMEM((CH, H), jnp.float32),
            pltpu.VMEM((CH, H), jnp.float32),
            pltpu.VMEM_SHARED((NP, H), jnp.float32),
            pltpu.SemaphoreType.DMA,
            pltpu.SemaphoreType.DMA,
        ],
    )
    def k(h_hbm, src_hbm, dst_hbm, em_hbm, out_hbm,
          src_v, dst_v, em_v, rows_a, rows_b, acc_sh, sem_a, sem_b):
        c = lax.axis_index("c")
        s = lax.axis_index("s")
        GR = CH // 16
        EMR = EP // 16

        _zero_rows(rows_a, CH, H)
        def zacc(t, carry):
            pltpu.sync_copy(rows_a, acc_sh.at[pl.ds(s * RPT + t * CH, CH)])
            return carry
        lax.fori_loop(0, RPT // CH, zacc, 0)
        plsc.subcore_barrier()

        pltpu.sync_copy(src_hbm.at[pl.ds(s * NCHW, NCHW)], src_v)
        pltpu.sync_copy(dst_hbm.at[pl.ds(s * NCHW, NCHW)], dst_v)
        pltpu.sync_copy(em_hbm.at[pl.ds(c * EMR + s * (EPW // 16), NCHW * GR)], em_v)

        off = c * NP
        def adj(i, carry):
            for f in range(CH // 16):
                src_v[i, pl.ds(f * 16, 16)] = src_v[i, pl.ds(f * 16, 16)] + off
            return carry
        lax.fori_loop(0, NCHW, adj, 0)

        def scale_scatter(j, rows_v):
            def group(g, gcarry):
                mv = em_v[j * GR + g]
                for l in range(16):
                    e = g * 16 + l
                    m = mv[l]
                    for f in range(H // 16):
                        rows_v[e, pl.ds(f * 16, 16)] = rows_v[e, pl.ds(f * 16, 16)] * m
                return gcarry

            lax.fori_loop(0, GR, group, 0)
            pltpu.sync_copy(rows_v, acc_sh.at[dst_v.at[j]], add=True)

        def pair(p, carry):
            cp_a = pltpu.async_copy(h_hbm.at[src_v.at[2 * p]], rows_a, sem_a)
            cp_b = pltpu.async_copy(h_hbm.at[src_v.at[2 * p + 1]], rows_b, sem_b)
            cp_a.wait()
            scale_scatter(2 * p, rows_a)
            cp_b.wait()
            scale_scatter(2 * p + 1, rows_b)
            return carry

        lax.fori_loop(0, NCHW // 2, pair, 0)
        plsc.subcore_barrier()

        def cout(t, carry):
            pltpu.sync_copy(acc_sh.at[pl.ds(s * RPT + t * CH, CH)], rows_a)
            pltpu.sync_copy(rows_a, out_hbm.at[pl.ds(c * NP + s * RPT + t * CH, CH)])
            return carry

        lax.fori_loop(0, RPT // CH, cout, 0)

    return k


def _tc_mlp(h, parts, W1, b1, W2, b2, epsv):
    din = h.shape[1]

    def body(h_ref, p_ref, W1_ref, b1_ref, W2_ref, b2_ref, eps_ref, o_ref):
        eps = eps_ref[0, 0]
        z = (1.0 + eps) * h_ref[...] + p_ref[0] + p_ref[1]
        y = jnp.maximum(jnp.dot(z, W1_ref[...], preferred_element_type=jnp.float32,
                        precision=lax.Precision.HIGHEST)
                        + b1_ref[...], 0.0)
        o_ref[...] = jnp.maximum(
            jnp.dot(y, W2_ref[...], preferred_element_type=jnp.float32,
                        precision=lax.Precision.HIGHEST)
            + b2_ref[...], 0.0)

    return pl.pallas_call(
        body,
        grid=(NB,),
        in_specs=[
            pl.BlockSpec((BLK, din), lambda i: (i, 0)),
            pl.BlockSpec((2, BLK, din), lambda i: (0, i, 0)),
            pl.BlockSpec((din, H), lambda i: (0, 0)),
            pl.BlockSpec((1, H), lambda i: (0, 0)),
            pl.BlockSpec((H, H), lambda i: (0, 0)),
            pl.BlockSpec((1, H), lambda i: (0, 0)),
            pl.BlockSpec((1, 128), lambda i: (0, 0)),
        ],
        out_specs=pl.BlockSpec((BLK, H), lambda i: (i, 0)),
        out_shape=jax.ShapeDtypeStruct((NP, H), jnp.float32),
    )(h, parts, W1, b1, W2, b2, epsv)


def _tc_mlp_pq(h, parts, W1, b1, W2, b2, epsv, Wp, Wq, b1pq):
    din = h.shape[1]

    def body(h_ref, p_ref, W1_ref, b1_ref, W2_ref, b2_ref, eps_ref,
             Wp_ref, Wq_ref, bpq_ref, z_ref, pp_ref, qq_ref):
        eps = eps_ref[0, 0]
        z = (1.0 + eps) * h_ref[...] + p_ref[0] + p_ref[1]
        y = jnp.maximum(jnp.dot(z, W1_ref[...], preferred_element_type=jnp.float32,
                        precision=lax.Precision.HIGHEST)
                        + b1_ref[...], 0.0)
        Z = jnp.maximum(jnp.dot(y, W2_ref[...], preferred_element_type=jnp.float32,
                        precision=lax.Precision.HIGHEST)
                        + b2_ref[...], 0.0)
        z_ref[...] = Z
        pp_ref[...] = jnp.dot(Z, Wp_ref[...], preferred_element_type=jnp.float32,
                        precision=lax.Precision.HIGHEST) + bpq_ref[...]
        qq_ref[...] = jnp.dot(Z, Wq_ref[...], preferred_element_type=jnp.float32,
                        precision=lax.Precision.HIGHEST) + bpq_ref[...]

    return pl.pallas_call(
        body,
        grid=(NB,),
        in_specs=[
            pl.BlockSpec((BLK, din), lambda i: (i, 0)),
            pl.BlockSpec((2, BLK, din), lambda i: (0, i, 0)),
            pl.BlockSpec((din, H), lambda i: (0, 0)),
            pl.BlockSpec((1, H), lambda i: (0, 0)),
            pl.BlockSpec((H, H), lambda i: (0, 0)),
            pl.BlockSpec((1, H), lambda i: (0, 0)),
            pl.BlockSpec((1, 128), lambda i: (0, 0)),
            pl.BlockSpec((H, 128), lambda i: (0, 0)),
            pl.BlockSpec((H, 128), lambda i: (0, 0)),
            pl.BlockSpec((1, 128), lambda i: (0, 0)),
        ],
        out_specs=[
            pl.BlockSpec((BLK, H), lambda i: (i, 0)),
            pl.BlockSpec((BLK, 128), lambda i: (i, 0)),
            pl.BlockSpec((BLK, 128), lambda i: (i, 0)),
        ],
        out_shape=[
            jax.ShapeDtypeStruct((NP, H), jnp.float32),
            jax.ShapeDtypeStruct((NP, 128), jnp.float32),
            jax.ShapeDtypeStruct((NP, 128), jnp.float32),
        ],
    )(h, parts, W1, b1, W2, b2, epsv, Wp, Wq, b1pq)


def _tc_expert_mlp(x, agg4, W1s, b1s, W2s, b2s, epsv):
    def body(x_ref, a_ref, W1_ref, b1_ref, W2_ref, b2_ref, eps_ref, o_ref):
        eps = eps_ref[0, 0, 0]
        agg = jnp.concatenate([a_ref[0, 0], a_ref[1, 0]], axis=-1)
        z = (1.0 + eps) * x_ref[...] + agg
        y = jnp.maximum(jnp.dot(z, W1_ref[0], preferred_element_type=jnp.float32,
                        precision=lax.Precision.HIGHEST)
                        + b1_ref[0], 0.0)
        o_ref[0] = jnp.maximum(
            jnp.dot(y, W2_ref[0], preferred_element_type=jnp.float32,
                        precision=lax.Precision.HIGHEST)
            + b2_ref[0], 0.0)

    return pl.pallas_call(
        body,
        grid=(2, NB),
        in_specs=[
            pl.BlockSpec((BLK, F), lambda k, i: (i, 0)),
            pl.BlockSpec((2, 1, BLK, H), lambda k, i: (0, k, i, 0)),
            pl.BlockSpec((1, F, H), lambda k, i: (k, 0, 0)),
            pl.BlockSpec((1, 1, H), lambda k, i: (k, 0, 0)),
            pl.BlockSpec((1, H, H), lambda k, i: (k, 0, 0)),
            pl.BlockSpec((1, 1, H), lambda k, i: (k, 0, 0)),
            pl.BlockSpec((1, 1, 128), lambda k, i: (k, 0, 0)),
        ],
        out_specs=pl.BlockSpec((1, BLK, H), lambda k, i: (k, i, 0)),
        out_shape=jax.ShapeDtypeStruct((2, NP, H), jnp.float32),
    )(x, agg4, W1s, b1s, W2s, b2s, epsv)


def _tc_final(h1e, agg5, batch3d, W1s, b1s, W2s, b2s, epsv, Wc1, bc1, Wc2, bc2):
    def body(h_ref, a_ref, b_ref, W1_ref, b1_ref, W2_ref, b2_ref, eps_ref,
             Wc1_ref, bc1_ref, Wc2_ref, bc2_ref, o_ref, hs_ref, cnt_ref):
        i = pl.program_id(1)
        eps = eps_ref[0, 0, 0]
        z = (1.0 + eps) * h_ref[0] + a_ref[0]
        y = jnp.maximum(jnp.dot(z, W1_ref[0], preferred_element_type=jnp.float32,
                        precision=lax.Precision.HIGHEST)
                        + b1_ref[0], 0.0)
        h2 = jnp.maximum(jnp.dot(y, W2_ref[0], preferred_element_type=jnp.float32,
                        precision=lax.Precision.HIGHEST)
                         + b2_ref[0], 0.0)
        b = b_ref[0, 0]
        gidx = lax.broadcasted_iota(jnp.int32, (G, BLK), 0)
        m = (b[None, :] == gidx).astype(jnp.float32)
        pooled = jnp.dot(m, h2, preferred_element_type=jnp.float32,
                        precision=lax.Precision.HIGHEST)
        csum = jnp.broadcast_to(jnp.sum(m, axis=1, keepdims=True), (G, 128))

        @pl.when(i == 0)
        def _():
            hs_ref[...] = pooled
            cnt_ref[...] = csum

        @pl.when(i > 0)
        def _():
            hs_ref[...] = hs_ref[...] + pooled
            cnt_ref[...] = cnt_ref[...] + csum

        @pl.when(i == NB - 1)
        def _():
            hg = hs_ref[...] / jnp.maximum(cnt_ref[...][:, :H], 1.0)
            o1 = jnp.maximum(
                jnp.dot(hg, Wc1_ref[0], preferred_element_type=jnp.float32,
                        precision=lax.Precision.HIGHEST)
                + bc1_ref[0], 0.0)
            o_ref[0] = jnp.dot(o1, Wc2_ref[0], preferred_element_type=jnp.float32,
                        precision=lax.Precision.HIGHEST) + bc2_ref[0]

    return pl.pallas_call(
        body,
        grid=(2, NB),
        in_specs=[
            pl.BlockSpec((1, BLK, H), lambda k, i: (k, i, 0)),
            pl.BlockSpec((1, BLK, H), lambda k, i: (k, i, 0)),
            pl.BlockSpec((1, 1, BLK), lambda k, i: (i, 0, 0)),
            pl.BlockSpec((1, H, H), lambda k, i: (k, 0, 0)),
            pl.BlockSpec((1, 1, H), lambda k, i: (k, 0, 0)),
            pl.BlockSpec((1, H, H), lambda k, i: (k, 0, 0)),
            pl.BlockSpec((1, 1, H), lambda k, i: (k, 0, 0)),
            pl.BlockSpec((1, 1, 128), lambda k, i: (k, 0, 0)),
            pl.BlockSpec((1, H, H), lambda k, i: (k, 0, 0)),
            pl.BlockSpec((1, 1, H), lambda k, i: (k, 0, 0)),
            pl.BlockSpec((1, H, 128), lambda k, i: (k, 0, 0)),
            pl.BlockSpec((1, 1, 128), lambda k, i: (k, 0, 0)),
        ],
        out_specs=pl.BlockSpec((1, G, 128), lambda k, i: (k, 0, 0)),
        out_shape=jax.ShapeDtypeStruct((2, G, 128), jnp.float32),
        scratch_shapes=[
            pltpu.VMEM((G, H), jnp.float32),
            pltpu.VMEM((G, 128), jnp.float32),
        ],
    )(h1e, agg5, batch3d, W1s, b1s, W2s, b2s, epsv, Wc1, bc1, Wc2, bc2)


def kernel(x, edge_index, batch, params):
    src = edge_index[0]
    dst = edge_index[1]
    E = src.shape[0]

    xp = jnp.pad(x, ((0, NP - N), (0, 0)))
    srcp = jnp.pad(src, (0, EP - E), constant_values=N).reshape(ECH, CH)
    dstp = jnp.pad(dst, (0, EP - E), constant_values=N).reshape(ECH, CH)
    batch3d = jnp.pad(batch, (0, NP - N), constant_values=G).reshape(NB, 1, BLK)

    cl0, cl1 = params["causal"]

    def epsv(e):
        return jnp.broadcast_to(e, (1, 128)).astype(jnp.float32)

    p1 = _sc_seg_unweighted(F)(xp, srcp, dstp).reshape(2, NP, F)
    h1 = _tc_mlp(xp, p1, cl0["W1"], cl0["b1"].reshape(1, H), cl0["W2"],
                 cl0["b2"].reshape(1, H), epsv(cl0["eps"]))
    p2 = _sc_seg_unweighted(H)(h1, srcp, dstp).reshape(2, NP, H)

    m0, m1 = params["mask"][0], params["mask"][1]
    Wp = jnp.concatenate([m0["W1"][:H], m1["W1"][:H]], axis=1)
    Wq = jnp.concatenate([m0["W1"][H:], m1["W1"][H:]], axis=1)
    b1pq = (jnp.concatenate([m0["b1"], m1["b1"]]) * 0.5).reshape(1, 128)
    Z, P, Q = _tc_mlp_pq(h1, p2, cl1["W1"], cl1["b1"].reshape(1, H), cl1["W2"],
                         cl1["b2"].reshape(1, H), epsv(cl1["eps"]), Wp, Wq, b1pq)

    w2all = jnp.stack([m0["W2"][:, 0], m1["W2"][:, 0]]) / TEMP
    b2all = jnp.broadcast_to(
        (jnp.stack([m0["b2"][0], m1["b2"][0]]) / TEMP)[:, None], (2, 16))

    em = _sc_mask()(P, Q, srcp, dstp, w2all, b2all)

    xh = jnp.concatenate([xp[:, :H], xp[:, H:]], axis=0)
    agg4 = _sc_expert0()(xh, srcp, dstp, em).reshape(2, 2, NP, H)

    enc = params["clf_enc"]
    W1s0 = jnp.stack([enc[k][0]["W1"] for k in range(2)])
    b1s0 = jnp.stack([enc[k][0]["b1"].reshape(1, H) for k in range(2)])
    W2s0 = jnp.stack([enc[k][0]["W2"] for k in range(2)])
    b2s0 = jnp.stack([enc[k][0]["b2"].reshape(1, H) for k in range(2)])
    eps0 = jnp.stack([jnp.broadcast_to(enc[k][0]["eps"], (1, 128)) for k in range(2)])
    h1e = _tc_expert_mlp(xp, agg4, W1s0, b1s0, W2s0, b2s0,
                         eps0.astype(jnp.float32))

    h1e_flat = h1e.reshape(2 * NP, H)
    agg5 = _sc_expert1()(h1e_flat, srcp, dstp, em).reshape(2, NP, H)

    W1s1 = jnp.stack([enc[k][1]["W1"] for k in range(2)])
    b1s1 = jnp.stack([enc[k][1]["b1"].reshape(1, H) for k in range(2)])
    W2s1 = jnp.stack([enc[k][1]["W2"] for k in range(2)])
    b2s1 = jnp.stack([enc[k][1]["b2"].reshape(1, H) for k in range(2)])
    eps1 = jnp.stack([jnp.broadcast_to(enc[k][1]["eps"], (1, 128)) for k in range(2)])

    hd = params["clf_head"]
    Wc1 = jnp.stack([hd[k]["W1"] for k in range(2)])
    bc1 = jnp.stack([hd[k]["b1"].reshape(1, H) for k in range(2)])
    Wc2 = jnp.stack([jnp.pad(hd[k]["W2"], ((0, 0), (0, 128 - C))) for k in range(2)])
    bc2 = jnp.stack([jnp.pad(jnp.broadcast_to(hd[k]["b2"], (C,)),
                             (0, 128 - C)).reshape(1, 128) for k in range(2)])

    out = _tc_final(h1e, agg5, batch3d, W1s1, b1s1, W2s1, b2s1,
                    eps1.astype(jnp.float32), Wc1, bc1, Wc2, bc2)
    return out[:, :, :C]
